# baseline (speedup 1.0000x reference)
import functools

import jax
import jax.numpy as jnp
from jax.experimental import pallas as pl
from jax.experimental.pallas import tpu as pltpu

F32 = jnp.float32
BF16 = jnp.bfloat16

EGOD = 64
MEMD = 128
NOBJD = 13
HH = 256
WW = 256
RPT = 8
PH = HH + 2 * RPT
PW = WW + 8
TPIX = RPT * PW
NT = PH // RPT
NI = NT - 2
NPIX = PH * PW
NVALID = float(HH * WW)
EPS = 1e-5
TSTEPS = 4


NP = RPT * WW


def _gru_conv1_body(feat_ref, mask_ref, wih_ref, whh_ref, bih_ref, bhh_ref,
                    tp_ref, ts_ref, y_ref, ps_ref, obs_ref, sbuf_ref):
    j = pl.program_id(0)
    xb = feat_ref[...].astype(BF16).reshape(TSTEPS * NP, EGOD)
    gi_all = jnp.dot(xb, wih_ref[...], preferred_element_type=F32)
    gi_all = gi_all.reshape(TSTEPS, NP, 3 * MEMD)
    m = mask_ref[...]
    bih = bih_ref[...]
    bhh = bhh_ref[...]
    state = jnp.zeros((NP, MEMD), F32)
    for t in range(TSTEPS):
        gh = jnp.dot(state.astype(BF16), whh_ref[...],
                     preferred_element_type=F32)
        gi = gi_all[t]
        r = 0.5 + 0.5 * jnp.tanh(0.5 * (gi[:, 0:MEMD] + bih[0:1]
                                        + gh[:, 0:MEMD] + bhh[0:1]))
        z = 0.5 + 0.5 * jnp.tanh(0.5 * (gi[:, MEMD:2 * MEMD] + bih[1:2]
                                        + gh[:, MEMD:2 * MEMD] + bhh[1:2]))
        n = jnp.tanh(gi[:, 2 * MEMD:] + bih[2:3]
                     + r * (gh[:, 2 * MEMD:] + bhh[2:3]))
        h_new = (1.0 - z) * n + z * state
        state = jnp.where(m[:, t:t + 1] > 0.0, h_new, state)
    state = jnp.where((j >= 1) & (j <= NI), state, 0.0)
    s3 = state.reshape(RPT, WW, MEMD)
    zc = jnp.zeros((RPT, 4, MEMD), F32)
    spad = jnp.concatenate([zc, s3, zc], axis=1).reshape(TPIX, MEMD)
    sbuf_ref[0:TPIX, :] = sbuf_ref[TPIX:2 * TPIX, :]
    sbuf_ref[TPIX:2 * TPIX, :] = sbuf_ref[2 * TPIX:3 * TPIX, :]
    sbuf_ref[2 * TPIX:3 * TPIX, :] = spad.astype(BF16)
    obs_ref[...] = jnp.sum(m, axis=1, keepdims=True).astype(jnp.int32)

    colok = _colok()
    acc = jnp.zeros((TPIX, 128), F32)
    pairs, solo = _tap_offsets(3)
    for i, (oa, ob) in enumerate(pairs):
        xp = jnp.concatenate([sbuf_ref[oa:oa + TPIX, :],
                              sbuf_ref[ob:ob + TPIX, :]], axis=1)
        acc = acc + jnp.dot(xp, tp_ref[i], preferred_element_type=F32)
    xs = sbuf_ref[solo:solo + TPIX, :]
    acc = acc + jnp.dot(xs, ts_ref[0], preferred_element_type=F32)
    y_ref[...] = acc
    accm = jnp.where(colok, acc, 0.0)
    ps_ref[0, 0, :] = jnp.sum(accm, axis=0)
    ps_ref[0, 1, :] = jnp.sum(accm * accm, axis=0)


def _colok():
    col = jax.lax.broadcasted_iota(jnp.int32, (TPIX, 1), 0) % PW
    return (col >= 4) & (col < 4 + WW)


def _bn_scale_shift(ps_ref, g_ref, b_ref, cin):
    ps = ps_ref[...]
    mu = jnp.sum(ps[:, 0, :], axis=0) * (1.0 / NVALID)
    msq = jnp.sum(ps[:, 1, :], axis=0) * (1.0 / NVALID)
    var = msq - mu * mu
    scale = (g_ref[...].reshape(1, cin)
             * jax.lax.rsqrt(var + EPS).reshape(1, cin))
    shift = b_ref[...].reshape(1, cin) - mu.reshape(1, cin) * scale
    return scale, shift


def _tap_offsets(khalf):
    offs = [TPIX + dh * PW + dw
            for dh in range(-khalf, khalf + 1)
            for dw in range(-khalf, khalf + 1)]
    pairs = [(offs[2 * i], offs[2 * i + 1]) for i in range(len(offs) // 2)]
    solo = offs[-1] if len(offs) % 2 else None
    return pairs, solo


def _conv_body(*refs, khalf, cin, cout, apply_bn):
    if apply_bn:
        (prev_ref, cur_ref, next_ref, ps_ref, g_ref, b_ref, tp_ref, ts_ref,
         y_ref, psout_ref) = refs
    else:
        (prev_ref, cur_ref, next_ref, tp_ref, ts_ref,
         y_ref, psout_ref) = refs
    j = pl.program_id(0)
    colok = _colok()
    if apply_bn:
        scale, shift = _bn_scale_shift(ps_ref, g_ref, b_ref, cin)

        def prep(x, rowok):
            x = jnp.maximum(x * scale + shift, 0.0)
            x = jnp.where(colok, x, 0.0)
            x = jnp.where(rowok, x, 0.0)
            return x.astype(BF16)

        xcat = jnp.concatenate(
            [prep(prev_ref[...], j > 0),
             prep(cur_ref[...], j >= 0),
             prep(next_ref[...], j < NI - 1)], axis=0)
    else:
        xcat = jnp.concatenate(
            [jnp.where(j > 0, prev_ref[...], 0.0),
             cur_ref[...],
             jnp.where(j < NI - 1, next_ref[...], 0.0)],
            axis=0).astype(BF16)
    acc = jnp.zeros((TPIX, cout), F32)
    pairs, solo = _tap_offsets(khalf)
    for i, (oa, ob) in enumerate(pairs):
        xa = jax.lax.slice_in_dim(xcat, oa, oa + TPIX, axis=0)
        xb = jax.lax.slice_in_dim(xcat, ob, ob + TPIX, axis=0)
        xp = jnp.concatenate([xa, xb], axis=1)
        acc = acc + jnp.dot(xp, tp_ref[i], preferred_element_type=F32)
    if solo is not None:
        xs = jax.lax.slice_in_dim(xcat, solo, solo + TPIX, axis=0)
        acc = acc + jnp.dot(xs, ts_ref[0], preferred_element_type=F32)
    y_ref[...] = acc
    accm = jnp.where(colok, acc, 0.0)
    psout_ref[0, 0, :] = jnp.sum(accm, axis=0)
    psout_ref[0, 1, :] = jnp.sum(accm * accm, axis=0)


def _head_body(x_ref, ps_ref, g_ref, b_ref, w_ref, bias_ref, out_ref):
    scale, shift = _bn_scale_shift(ps_ref, g_ref, b_ref, 48)
    xn = jnp.maximum(x_ref[...] * scale + shift, 0.0).astype(BF16)
    out_ref[...] = (jnp.dot(xn, w_ref[...], preferred_element_type=F32)
                    + bias_ref[...])


def _conv_layer(x, ps, g, b, taps, cin, cout, khalf, apply_bn):
    body = functools.partial(_conv_body, khalf=khalf, cin=cin, cout=cout,
                             apply_bn=apply_bn)
    nk = (2 * khalf + 1) ** 2
    npair = nk // 2
    tpair = taps[:2 * npair].reshape(npair, 2 * cin, cout)
    tsolo = taps[2 * npair:] if nk % 2 else taps[:1]
    in_specs = [
        pl.BlockSpec((TPIX, cin), lambda j: (j, 0)),
        pl.BlockSpec((TPIX, cin), lambda j: (j + 1, 0)),
        pl.BlockSpec((TPIX, cin), lambda j: (j + 2, 0)),
    ]
    ins = [x, x, x]
    if apply_bn:
        in_specs += [
            pl.BlockSpec((NI, 8, cin), lambda j: (0, 0, 0)),
            pl.BlockSpec((1, cin), lambda j: (0, 0)),
            pl.BlockSpec((1, cin), lambda j: (0, 0)),
        ]
        ins += [ps, g.reshape(1, cin), b.reshape(1, cin)]
    in_specs.append(pl.BlockSpec((npair, 2 * cin, cout), lambda j: (0, 0, 0)))
    ins.append(tpair)
    in_specs.append(pl.BlockSpec((1, cin, cout), lambda j: (0, 0, 0)))
    ins.append(tsolo)
    y, psout = pl.pallas_call(
        body,
        grid=(NI,),
        in_specs=in_specs,
        out_specs=[pl.BlockSpec((TPIX, cout), lambda j: (j + 1, 0)),
                   pl.BlockSpec((1, 8, cout), lambda j: (j, 0, 0))],
        out_shape=[jax.ShapeDtypeStruct((NPIX, cout), F32),
                   jax.ShapeDtypeStruct((NI, 8, cout), F32)],
    )(*ins)
    return y, psout


def _taps(c, cin, cout):
    k = c.shape[2]
    return c.transpose(2, 3, 1, 0).reshape(k * k, cin, cout).astype(BF16)


def kernel(features, masks_inliers, w_ih, w_hh, b_ih, b_hh,
           c1, g1, be1, c2, g2, be2, c3, g3, be3, c4, g4, be4, c5, c5b):
    mp = masks_inliers.reshape(TSTEPS, HH * WW).T.astype(F32)

    taps1 = _taps(c1, MEMD, 128)
    tpair1 = taps1[:48].reshape(24, 2 * MEMD, 128)
    tsolo1 = taps1[48:]
    fidx = lambda j: jnp.clip(j - 1, 0, NI - 1)
    y1, ps1, obs = pl.pallas_call(
        _gru_conv1_body,
        grid=(NT,),
        in_specs=[
            pl.BlockSpec((1, TSTEPS, RPT, WW, EGOD),
                         lambda j: (0, 0, fidx(j), 0, 0)),
            pl.BlockSpec((NP, TSTEPS), lambda j: (fidx(j), 0)),
            pl.BlockSpec((EGOD, 3 * MEMD), lambda j: (0, 0)),
            pl.BlockSpec((MEMD, 3 * MEMD), lambda j: (0, 0)),
            pl.BlockSpec((3, MEMD), lambda j: (0, 0)),
            pl.BlockSpec((3, MEMD), lambda j: (0, 0)),
            pl.BlockSpec((24, 2 * MEMD, 128), lambda j: (0, 0, 0)),
            pl.BlockSpec((1, MEMD, 128), lambda j: (0, 0, 0)),
        ],
        out_specs=[
            pl.BlockSpec((TPIX, 128), lambda j: (jnp.clip(j - 1, 0, NI), 0)),
            pl.BlockSpec((1, 8, 128), lambda j: (jnp.clip(j - 2, 0, NI - 1),
                                                 0, 0)),
            pl.BlockSpec((NP, 1), lambda j: (fidx(j), 0)),
        ],
        out_shape=[jax.ShapeDtypeStruct((NPIX, 128), F32),
                   jax.ShapeDtypeStruct((NI, 8, 128), F32),
                   jax.ShapeDtypeStruct((HH * WW, 1), jnp.int32)],
        scratch_shapes=[pltpu.VMEM((3 * TPIX, MEMD), BF16)],
    )(features, mp,
      w_ih.T.astype(BF16), w_hh.T.astype(BF16),
      b_ih.reshape(3, MEMD), b_hh.reshape(3, MEMD),
      tpair1, tsolo1)
    y2, ps2 = _conv_layer(y1, ps1, g1, be1, _taps(c2, 128, 64),
                          128, 64, 1, True)
    y3, ps3 = _conv_layer(y2, ps2, g2, be2, _taps(c3, 64, 48),
                          64, 48, 1, True)
    y4, ps4 = _conv_layer(y3, ps3, g3, be3, _taps(c4, 48, 48),
                          48, 48, 1, True)

    y5 = pl.pallas_call(
        _head_body,
        grid=(NI,),
        in_specs=[
            pl.BlockSpec((TPIX, 48), lambda j: (j + 1, 0)),
            pl.BlockSpec((NI, 8, 48), lambda j: (0, 0, 0)),
            pl.BlockSpec((1, 48), lambda j: (0, 0)),
            pl.BlockSpec((1, 48), lambda j: (0, 0)),
            pl.BlockSpec((48, NOBJD), lambda j: (0, 0)),
            pl.BlockSpec((1, NOBJD), lambda j: (0, 0)),
        ],
        out_specs=pl.BlockSpec((TPIX, NOBJD), lambda j: (j + 1, 0)),
        out_shape=jax.ShapeDtypeStruct((NPIX, NOBJD), F32),
    )(y4, ps4, g4.reshape(1, 48), be4.reshape(1, 48),
      c5.reshape(NOBJD, 48).T.astype(BF16), c5b.reshape(1, NOBJD))

    semmap = y5.reshape(PH, PW, NOBJD)[RPT:RPT + HH, 4:4 + WW, :]
    semmap = semmap.transpose(2, 0, 1)[None]
    observed = obs.reshape(1, HH, WW)
    return (semmap, observed)


# speedup vs baseline: 1.1797x; 1.1797x over previous
import functools

import jax
import jax.numpy as jnp
from jax.experimental import pallas as pl
from jax.experimental.pallas import tpu as pltpu

F32 = jnp.float32
BF16 = jnp.bfloat16

EGOD = 64
MEMD = 128
NOBJD = 13
HH = 256
WW = 256
RPT = 16
PH = HH + 2 * RPT
PW = WW + 8
TPIX = RPT * PW
NT = PH // RPT
NI = NT - 2
NPIX = PH * PW
NVALID = float(HH * WW)
EPS = 1e-5
TSTEPS = 4


NP = RPT * WW


def _gru_body(feat_ref, mask_ref, wih_ref, whh_ref, bih_ref, bhh_ref,
              state_ref, obs_ref):
    xb = feat_ref[...].astype(BF16).reshape(TSTEPS * NP, EGOD)
    gi_all = jnp.dot(xb, wih_ref[...], preferred_element_type=F32)
    gi_all = gi_all.reshape(TSTEPS, NP, 3 * MEMD)
    m = mask_ref[...]
    bih = bih_ref[...]
    bhh = bhh_ref[...]
    state = jnp.zeros((NP, MEMD), F32)
    for t in range(TSTEPS):
        gh = jnp.dot(state.astype(BF16), whh_ref[...],
                     preferred_element_type=F32)
        gi = gi_all[t]
        r = 0.5 + 0.5 * jnp.tanh(0.5 * (gi[:, 0:MEMD] + bih[0:1]
                                        + gh[:, 0:MEMD] + bhh[0:1]))
        z = 0.5 + 0.5 * jnp.tanh(0.5 * (gi[:, MEMD:2 * MEMD] + bih[1:2]
                                        + gh[:, MEMD:2 * MEMD] + bhh[1:2]))
        n = jnp.tanh(gi[:, 2 * MEMD:] + bih[2:3]
                     + r * (gh[:, 2 * MEMD:] + bhh[2:3]))
        h_new = (1.0 - z) * n + z * state
        state = jnp.where(m[:, t:t + 1] > 0.0, h_new, state)
    s3 = state.reshape(RPT, WW, MEMD)
    zc = jnp.zeros((RPT, 4, MEMD), F32)
    state_ref[...] = jnp.concatenate([zc, s3, zc], axis=1).reshape(TPIX, MEMD)
    obs_ref[...] = jnp.sum(m, axis=1, keepdims=True).astype(jnp.int32)


def _colok():
    col = jax.lax.broadcasted_iota(jnp.int32, (TPIX, 1), 0) % PW
    return (col >= 4) & (col < 4 + WW)


def _bn_scale_shift(ps_ref, g_ref, b_ref, cin):
    ps = ps_ref[...]
    mu = jnp.sum(ps[:, 0, :], axis=0) * (1.0 / NVALID)
    msq = jnp.sum(ps[:, 1, :], axis=0) * (1.0 / NVALID)
    var = msq - mu * mu
    scale = (g_ref[...].reshape(1, cin)
             * jax.lax.rsqrt(var + EPS).reshape(1, cin))
    shift = b_ref[...].reshape(1, cin) - mu.reshape(1, cin) * scale
    return scale, shift


def _tap_offsets(khalf):
    offs = [TPIX + dh * PW + dw
            for dh in range(-khalf, khalf + 1)
            for dw in range(-khalf, khalf + 1)]
    pairs = [(offs[2 * i], offs[2 * i + 1]) for i in range(len(offs) // 2)]
    solo = offs[-1] if len(offs) % 2 else None
    return pairs, solo


def _conv_body(*refs, khalf, cin, cout, apply_bn):
    if apply_bn:
        (prev_ref, cur_ref, next_ref, ps_ref, g_ref, b_ref, tp_ref, ts_ref,
         y_ref, psout_ref) = refs
    else:
        (prev_ref, cur_ref, next_ref, tp_ref, ts_ref,
         y_ref, psout_ref) = refs
    j = pl.program_id(0)
    colok = _colok()
    if apply_bn:
        scale, shift = _bn_scale_shift(ps_ref, g_ref, b_ref, cin)

        def prep(x, rowok):
            x = jnp.maximum(x * scale + shift, 0.0)
            x = jnp.where(colok, x, 0.0)
            x = jnp.where(rowok, x, 0.0)
            return x.astype(BF16)

        xcat = jnp.concatenate(
            [prep(prev_ref[...], j > 0),
             prep(cur_ref[...], j >= 0),
             prep(next_ref[...], j < NI - 1)], axis=0)
    else:
        xcat = jnp.concatenate(
            [jnp.where(j > 0, prev_ref[...], 0.0),
             cur_ref[...],
             jnp.where(j < NI - 1, next_ref[...], 0.0)],
            axis=0).astype(BF16)
    acc = jnp.zeros((TPIX, cout), F32)
    pairs, solo = _tap_offsets(khalf)
    for i, (oa, ob) in enumerate(pairs):
        xa = jax.lax.slice_in_dim(xcat, oa, oa + TPIX, axis=0)
        xb = jax.lax.slice_in_dim(xcat, ob, ob + TPIX, axis=0)
        xp = jnp.concatenate([xa, xb], axis=1)
        acc = acc + jnp.dot(xp, tp_ref[i], preferred_element_type=F32)
    if solo is not None:
        xs = jax.lax.slice_in_dim(xcat, solo, solo + TPIX, axis=0)
        acc = acc + jnp.dot(xs, ts_ref[0], preferred_element_type=F32)
    y_ref[...] = acc
    accm = jnp.where(colok, acc, 0.0)
    psout_ref[0, 0, :] = jnp.sum(accm, axis=0)
    psout_ref[0, 1, :] = jnp.sum(accm * accm, axis=0)


def _head_body(x_ref, ps_ref, g_ref, b_ref, w_ref, bias_ref, out_ref):
    scale, shift = _bn_scale_shift(ps_ref, g_ref, b_ref, 48)
    xn = jnp.maximum(x_ref[...] * scale + shift, 0.0).astype(BF16)
    out_ref[...] = (jnp.dot(xn, w_ref[...], preferred_element_type=F32)
                    + bias_ref[...])


def _conv_layer(x, ps, g, b, taps, cin, cout, khalf, apply_bn):
    body = functools.partial(_conv_body, khalf=khalf, cin=cin, cout=cout,
                             apply_bn=apply_bn)
    nk = (2 * khalf + 1) ** 2
    npair = nk // 2
    tpair = taps[:2 * npair].reshape(npair, 2 * cin, cout)
    tsolo = taps[2 * npair:] if nk % 2 else taps[:1]
    in_specs = [
        pl.BlockSpec((TPIX, cin), lambda j: (j, 0)),
        pl.BlockSpec((TPIX, cin), lambda j: (j + 1, 0)),
        pl.BlockSpec((TPIX, cin), lambda j: (j + 2, 0)),
    ]
    ins = [x, x, x]
    if apply_bn:
        in_specs += [
            pl.BlockSpec((NI, 8, cin), lambda j: (0, 0, 0)),
            pl.BlockSpec((1, cin), lambda j: (0, 0)),
            pl.BlockSpec((1, cin), lambda j: (0, 0)),
        ]
        ins += [ps, g.reshape(1, cin), b.reshape(1, cin)]
    in_specs.append(pl.BlockSpec((npair, 2 * cin, cout), lambda j: (0, 0, 0)))
    ins.append(tpair)
    in_specs.append(pl.BlockSpec((1, cin, cout), lambda j: (0, 0, 0)))
    ins.append(tsolo)
    y, psout = pl.pallas_call(
        body,
        grid=(NI,),
        in_specs=in_specs,
        out_specs=[pl.BlockSpec((TPIX, cout), lambda j: (j + 1, 0)),
                   pl.BlockSpec((1, 8, cout), lambda j: (j, 0, 0))],
        out_shape=[jax.ShapeDtypeStruct((NPIX, cout), F32),
                   jax.ShapeDtypeStruct((NI, 8, cout), F32)],
    )(*ins)
    return y, psout


def _taps(c, cin, cout):
    k = c.shape[2]
    return c.transpose(2, 3, 1, 0).reshape(k * k, cin, cout).astype(BF16)


def kernel(features, masks_inliers, w_ih, w_hh, b_ih, b_hh,
           c1, g1, be1, c2, g2, be2, c3, g3, be3, c4, g4, be4, c5, c5b):
    mp = masks_inliers.reshape(TSTEPS, HH * WW).T.astype(F32)

    state, obs = pl.pallas_call(
        _gru_body,
        grid=(NI,),
        in_specs=[
            pl.BlockSpec((1, TSTEPS, RPT, WW, EGOD),
                         lambda j: (0, 0, j, 0, 0)),
            pl.BlockSpec((NP, TSTEPS), lambda j: (j, 0)),
            pl.BlockSpec((EGOD, 3 * MEMD), lambda j: (0, 0)),
            pl.BlockSpec((MEMD, 3 * MEMD), lambda j: (0, 0)),
            pl.BlockSpec((3, MEMD), lambda j: (0, 0)),
            pl.BlockSpec((3, MEMD), lambda j: (0, 0)),
        ],
        out_specs=[pl.BlockSpec((TPIX, MEMD), lambda j: (j + 1, 0)),
                   pl.BlockSpec((NP, 1), lambda j: (j, 0))],
        out_shape=[jax.ShapeDtypeStruct((NPIX, MEMD), F32),
                   jax.ShapeDtypeStruct((HH * WW, 1), jnp.int32)],
    )(features, mp,
      w_ih.T.astype(BF16), w_hh.T.astype(BF16),
      b_ih.reshape(3, MEMD), b_hh.reshape(3, MEMD))

    y1, ps1 = _conv_layer(state, None, None, None, _taps(c1, MEMD, 128),
                          MEMD, 128, 3, False)
    y2, ps2 = _conv_layer(y1, ps1, g1, be1, _taps(c2, 128, 64),
                          128, 64, 1, True)
    y3, ps3 = _conv_layer(y2, ps2, g2, be2, _taps(c3, 64, 48),
                          64, 48, 1, True)
    y4, ps4 = _conv_layer(y3, ps3, g3, be3, _taps(c4, 48, 48),
                          48, 48, 1, True)

    y5 = pl.pallas_call(
        _head_body,
        grid=(NI,),
        in_specs=[
            pl.BlockSpec((TPIX, 48), lambda j: (j + 1, 0)),
            pl.BlockSpec((NI, 8, 48), lambda j: (0, 0, 0)),
            pl.BlockSpec((1, 48), lambda j: (0, 0)),
            pl.BlockSpec((1, 48), lambda j: (0, 0)),
            pl.BlockSpec((48, NOBJD), lambda j: (0, 0)),
            pl.BlockSpec((1, NOBJD), lambda j: (0, 0)),
        ],
        out_specs=pl.BlockSpec((TPIX, NOBJD), lambda j: (j + 1, 0)),
        out_shape=jax.ShapeDtypeStruct((NPIX, NOBJD), F32),
    )(y4, ps4, g4.reshape(1, 48), be4.reshape(1, 48),
      c5.reshape(NOBJD, 48).T.astype(BF16), c5b.reshape(1, NOBJD))

    semmap = y5.reshape(PH, PW, NOBJD)[RPT:RPT + HH, 4:4 + WW, :]
    semmap = semmap.transpose(2, 0, 1)[None]
    observed = obs.reshape(1, HH, WW)
    return (semmap, observed)
